# baseline (speedup 1.0000x reference)
import functools

import jax
import jax.numpy as jnp
import numpy as np
from jax import lax
from jax.experimental import pallas as pl
from jax.experimental.pallas import tpu as pltpu
from jax.experimental.pallas import tpu_sc as plsc


def _gather_body(consts, ej, ew, idxj, idxw, outj, outw,
                 idxj_v, idxw_v, rowsj_v, rowsw_v, semj, semw):
    (per_w, t_steps, c_rows) = consts
    cid = lax.axis_index("c")
    sid = lax.axis_index("s")
    wid = sid * 2 + cid
    pltpu.sync_copy(idxj.at[wid], idxj_v)
    pltpu.sync_copy(idxw.at[wid], idxw_v)

    def step(t, carry):
        base = wid * per_w + t * c_rows
        cpj = pltpu.async_copy(ej.at[idxj_v.at[t]], rowsj_v, semj)
        cpw = pltpu.async_copy(ew.at[idxw_v.at[t]], rowsw_v, semw)
        cpj.wait()
        cpw.wait()
        pltpu.sync_copy(rowsj_v, outj.at[pl.ds(base, c_rows)])
        pltpu.sync_copy(rowsw_v, outw.at[pl.ds(base, c_rows)])
        return carry

    lax.fori_loop(0, t_steps, step, 0)


def _attn_body(bn, k, gj_ref, gwp_ref, vj_ref, vw_ref, ev_ref, w1a_ref,
               w1b_ref, w2_ref, b_ref, v_ref, out_ref):
    gj = gj_ref[...]
    gwp = gwp_ref[...]
    dw = w1b_ref.shape[0]
    f_dim = gj.shape[-1]
    base = jnp.dot(ev_ref[...], w1a_ref[...],
                   preferred_element_type=jnp.float32) + b_ref[...]
    avj = jnp.dot(gj, w2_ref[...], preferred_element_type=jnp.float32)
    avw = jnp.concatenate(
        [jnp.dot(gwp[:, g * dw:(g + 1) * dw], w1b_ref[...],
                 preferred_element_type=jnp.float32) for g in range(8)],
        axis=0)
    a_dim = avj.shape[-1]
    mj = (vj_ref[...] > 0).astype(jnp.float32)
    mw = (vw_ref[...] > 0).astype(jnp.float32)
    av3 = (base[:, None, :]
           + mj[:, :, None] * avj.reshape(bn, k, a_dim)
           + mw[:, :, None] * avw.reshape(bn, k, a_dim))
    r = jnp.maximum(av3, 0.0)
    x = jnp.sum(r * v_ref[...][None], axis=2)
    x = x - jnp.max(x, axis=1, keepdims=True)
    e = jnp.exp(x)
    a = e / jnp.sum(e, axis=1, keepdims=True)
    am = a * mj
    gj3 = gj.reshape(bn, k, f_dim)
    out_ref[...] = jnp.sum(am[:, :, None] * gj3, axis=1)


def kernel(ev, ej, ew, v_j, v_w, W_1, W_2, b, v):
    n, f_dim = ev.shape
    k = v_j.shape[1]
    dw_dim = ew.shape[1]
    a_dim = W_1.shape[1]
    e_rows = n * k
    pack = 128 // dw_dim

    w1a = W_1[:f_dim]
    w1b = W_1[f_dim:]
    bn = 200
    assert n % bn == 0
    nblk = n // bn
    rb = bn * k

    info = plsc.get_sparse_core_info()
    nw = info.num_cores * info.num_subcores
    n_chunks = 10
    nc = n // n_chunks
    ec = nc * k
    assert ec % nw == 0
    per_w = ec // nw
    c_rows = 40
    assert per_w % c_rows == 0
    t_steps = per_w // c_rows
    assert nc % bn == 0
    nblk_c = nc // bn

    jc = jnp.maximum(v_j - 1, 0)
    wc = jnp.maximum(v_w - 1, 0)
    wcp = jnp.transpose(wc.reshape(n // bn, pack, rb // pack), (0, 2, 1))

    mesh = plsc.VectorSubcoreMesh(core_axis_name="c", subcore_axis_name="s")
    gather = pl.kernel(
        functools.partial(_gather_body, (per_w, t_steps, c_rows)),
        out_type=[jax.ShapeDtypeStruct((ec, f_dim), jnp.float32),
                  jax.ShapeDtypeStruct((ec, dw_dim), jnp.float32)],
        mesh=mesh,
        scratch_types=[
            pltpu.VMEM((t_steps, c_rows), jnp.int32),
            pltpu.VMEM((t_steps, c_rows), jnp.int32),
            pltpu.VMEM((c_rows, f_dim), jnp.float32),
            pltpu.VMEM((c_rows, dw_dim), jnp.float32),
            pltpu.SemaphoreType.DMA,
            pltpu.SemaphoreType.DMA,
        ],
        compiler_params=pltpu.CompilerParams(use_tc_tiling_on_sc=False),
    )

    attn = pl.pallas_call(
        functools.partial(_attn_body, bn, k),
        grid=(nblk_c,),
        in_specs=[
            pl.BlockSpec((rb, f_dim), lambda i: (i, 0)),
            pl.BlockSpec((rb // pack, 128), lambda i: (i, 0)),
            pl.BlockSpec((bn, k), lambda i: (i, 0)),
            pl.BlockSpec((bn, k), lambda i: (i, 0)),
            pl.BlockSpec((bn, f_dim), lambda i: (i, 0)),
            pl.BlockSpec((f_dim, a_dim), lambda i: (0, 0)),
            pl.BlockSpec((dw_dim, a_dim), lambda i: (0, 0)),
            pl.BlockSpec((f_dim, a_dim), lambda i: (0, 0)),
            pl.BlockSpec((1, a_dim), lambda i: (0, 0)),
            pl.BlockSpec((1, a_dim), lambda i: (0, 0)),
        ],
        out_specs=pl.BlockSpec((bn, f_dim), lambda i: (i, 0)),
        out_shape=jax.ShapeDtypeStruct((nc, f_dim), jnp.float32),
    )

    outs = []
    for s in range(n_chunks):
        nd = slice(s * nc, (s + 1) * nc)
        idxj_s = jc[nd].reshape(nw, t_steps, c_rows)
        idxw_s = wcp[s * nblk_c:(s + 1) * nblk_c].reshape(nw, t_steps, c_rows)
        gj, gw = gather(ej, ew, idxj_s, idxw_s)
        gwp = gw.reshape(ec // pack, 128)
        outs.append(attn(gj, gwp, v_j[nd], v_w[nd], ev[nd],
                         w1a, w1b, W_2, b, v))
    return jnp.concatenate(outs, axis=0)


# speedup vs baseline: 1.0588x; 1.0588x over previous
import functools

import jax
import jax.numpy as jnp
import numpy as np
from jax import lax
from jax.experimental import pallas as pl
from jax.experimental.pallas import tpu as pltpu
from jax.experimental.pallas import tpu_sc as plsc


def _gather_body(consts, ej, ew, idxj, idxw, outj, outw,
                 idxj_v, idxw_v, rj0, rj1, rw0, rw1,
                 semgj, semgw, semwj, semww):
    (per_w, t_steps, c_rows) = consts
    cid = lax.axis_index("c")
    sid = lax.axis_index("s")
    wid = sid * 2 + cid
    pltpu.sync_copy(idxj.at[wid], idxj_v)
    pltpu.sync_copy(idxw.at[wid], idxw_v)

    rjs = (rj0, rj1)
    rws = (rw0, rw1)
    gj_cp = [None] * t_steps
    gw_cp = [None] * t_steps
    wj_cp = [None] * t_steps
    ww_cp = [None] * t_steps
    gj_cp[0] = pltpu.async_copy(ej.at[idxj_v.at[0]], rjs[0], semgj)
    gw_cp[0] = pltpu.async_copy(ew.at[idxw_v.at[0]], rws[0], semgw)
    for t in range(t_steps):
        b = t % 2
        if t >= 1:
            wj_cp[t - 1].wait()
            ww_cp[t - 1].wait()
        if t + 1 < t_steps:
            nb = (t + 1) % 2
            gj_cp[t + 1] = pltpu.async_copy(ej.at[idxj_v.at[t + 1]],
                                            rjs[nb], semgj)
            gw_cp[t + 1] = pltpu.async_copy(ew.at[idxw_v.at[t + 1]],
                                            rws[nb], semgw)
        gj_cp[t].wait()
        gw_cp[t].wait()
        base = wid * per_w + t * c_rows
        wj_cp[t] = pltpu.async_copy(rjs[b], outj.at[pl.ds(base, c_rows)],
                                    semwj)
        ww_cp[t] = pltpu.async_copy(rws[b], outw.at[pl.ds(base, c_rows)],
                                    semww)
    wj_cp[t_steps - 1].wait()
    ww_cp[t_steps - 1].wait()


def _attn_body(bn, k, gj_ref, gwp_ref, vj_ref, vw_ref, ev_ref, w1a_ref,
               w1b_ref, w2_ref, b_ref, v_ref, out_ref):
    gj = gj_ref[...]
    gwp = gwp_ref[...]
    dw = w1b_ref.shape[0]
    f_dim = gj.shape[-1]
    base = jnp.dot(ev_ref[...], w1a_ref[...],
                   preferred_element_type=jnp.float32) + b_ref[...]
    avj = jnp.dot(gj, w2_ref[...], preferred_element_type=jnp.float32)
    avw = jnp.concatenate(
        [jnp.dot(gwp[:, g * dw:(g + 1) * dw], w1b_ref[...],
                 preferred_element_type=jnp.float32) for g in range(8)],
        axis=0)
    a_dim = avj.shape[-1]
    mj = (vj_ref[...] > 0).astype(jnp.float32)
    mw = (vw_ref[...] > 0).astype(jnp.float32)
    av3 = (base[:, None, :]
           + mj[:, :, None] * avj.reshape(bn, k, a_dim)
           + mw[:, :, None] * avw.reshape(bn, k, a_dim))
    r = jnp.maximum(av3, 0.0)
    x = jnp.sum(r * v_ref[...][None], axis=2)
    x = x - jnp.max(x, axis=1, keepdims=True)
    e = jnp.exp(x)
    a = e / jnp.sum(e, axis=1, keepdims=True)
    am = a * mj
    gj3 = gj.reshape(bn, k, f_dim)
    out_ref[...] = jnp.sum(am[:, :, None] * gj3, axis=1)


def kernel(ev, ej, ew, v_j, v_w, W_1, W_2, b, v):
    n, f_dim = ev.shape
    k = v_j.shape[1]
    dw_dim = ew.shape[1]
    a_dim = W_1.shape[1]
    e_rows = n * k
    pack = 128 // dw_dim

    w1a = W_1[:f_dim]
    w1b = W_1[f_dim:]
    bn = 400
    assert n % bn == 0
    nblk = n // bn
    rb = bn * k

    info = plsc.get_sparse_core_info()
    nw = info.num_cores * info.num_subcores
    n_chunks = 5
    nc = n // n_chunks
    ec = nc * k
    assert ec % nw == 0
    per_w = ec // nw
    c_rows = 80
    assert per_w % c_rows == 0
    t_steps = per_w // c_rows
    assert nc % bn == 0
    nblk_c = nc // bn

    jc = jnp.maximum(v_j - 1, 0)
    wc = jnp.maximum(v_w - 1, 0)
    wcp = jnp.transpose(wc.reshape(n // bn, pack, rb // pack), (0, 2, 1))

    mesh = plsc.VectorSubcoreMesh(core_axis_name="c", subcore_axis_name="s")
    gather = pl.kernel(
        functools.partial(_gather_body, (per_w, t_steps, c_rows)),
        out_type=[jax.ShapeDtypeStruct((ec, f_dim), jnp.float32),
                  jax.ShapeDtypeStruct((ec, dw_dim), jnp.float32)],
        mesh=mesh,
        scratch_types=[
            pltpu.VMEM((t_steps, c_rows), jnp.int32),
            pltpu.VMEM((t_steps, c_rows), jnp.int32),
            pltpu.VMEM((c_rows, f_dim), jnp.float32),
            pltpu.VMEM((c_rows, f_dim), jnp.float32),
            pltpu.VMEM((c_rows, dw_dim), jnp.float32),
            pltpu.VMEM((c_rows, dw_dim), jnp.float32),
            pltpu.SemaphoreType.DMA,
            pltpu.SemaphoreType.DMA,
            pltpu.SemaphoreType.DMA,
            pltpu.SemaphoreType.DMA,
        ],
        compiler_params=pltpu.CompilerParams(use_tc_tiling_on_sc=False),
    )

    attn = pl.pallas_call(
        functools.partial(_attn_body, bn, k),
        grid=(nblk_c,),
        in_specs=[
            pl.BlockSpec((rb, f_dim), lambda i: (i, 0)),
            pl.BlockSpec((rb // pack, 128), lambda i: (i, 0)),
            pl.BlockSpec((bn, k), lambda i: (i, 0)),
            pl.BlockSpec((bn, k), lambda i: (i, 0)),
            pl.BlockSpec((bn, f_dim), lambda i: (i, 0)),
            pl.BlockSpec((f_dim, a_dim), lambda i: (0, 0)),
            pl.BlockSpec((dw_dim, a_dim), lambda i: (0, 0)),
            pl.BlockSpec((f_dim, a_dim), lambda i: (0, 0)),
            pl.BlockSpec((1, a_dim), lambda i: (0, 0)),
            pl.BlockSpec((1, a_dim), lambda i: (0, 0)),
        ],
        out_specs=pl.BlockSpec((bn, f_dim), lambda i: (i, 0)),
        out_shape=jax.ShapeDtypeStruct((nc, f_dim), jnp.float32),
    )

    outs = []
    for s in range(n_chunks):
        nd = slice(s * nc, (s + 1) * nc)
        idxj_s = jc[nd].reshape(nw, t_steps, c_rows)
        idxw_s = wcp[s * nblk_c:(s + 1) * nblk_c].reshape(nw, t_steps, c_rows)
        gj, gw = gather(ej, ew, idxj_s, idxw_s)
        gwp = gw.reshape(ec // pack, 128)
        outs.append(attn(gj, gwp, v_j[nd], v_w[nd], ev[nd],
                         w1a, w1b, W_2, b, v))
    return jnp.concatenate(outs, axis=0)


# speedup vs baseline: 1.0852x; 1.0249x over previous
import functools

import jax
import jax.numpy as jnp
import numpy as np
from jax import lax
from jax.experimental import pallas as pl
from jax.experimental.pallas import tpu as pltpu
from jax.experimental.pallas import tpu_sc as plsc


def _gather_body(consts, ej, ew, idxj, idxw, outj, outw,
                 idxj_v, idxw_v, rj0, rj1, rw0, rw1,
                 semgj, semgw, semwj, semww):
    (per_w, t_steps, c_rows) = consts
    cid = lax.axis_index("c")
    sid = lax.axis_index("s")
    wid = sid * 2 + cid
    pltpu.sync_copy(idxj.at[wid], idxj_v)
    pltpu.sync_copy(idxw.at[wid], idxw_v)

    del rj1, rw1, semwj, semww

    def step(t, carry):
        base = wid * per_w + t * c_rows
        cpj = pltpu.async_copy(ej.at[idxj_v.at[t]], rj0, semgj)
        cpw = pltpu.async_copy(ew.at[idxw_v.at[t]], rw0, semgw)
        cpj.wait()
        cpw.wait()
        pltpu.sync_copy(rj0, outj.at[pl.ds(base, c_rows)])
        pltpu.sync_copy(rw0, outw.at[pl.ds(base, c_rows)])
        return carry

    lax.fori_loop(0, t_steps, step, 0)


def _attn_body(bn, k, gj_ref, gwp_ref, vj_ref, vw_ref, ev_ref, w1a_ref,
               w1b_ref, w2_ref, b_ref, v_ref, out_ref):
    gj = gj_ref[...]
    gwp = gwp_ref[...]
    dw = w1b_ref.shape[0]
    f_dim = gj.shape[-1]
    base = jnp.dot(ev_ref[...], w1a_ref[...],
                   preferred_element_type=jnp.float32) + b_ref[...]
    avj = jnp.dot(gj, w2_ref[...], preferred_element_type=jnp.float32)
    avw = jnp.concatenate(
        [jnp.dot(gwp[:, g * dw:(g + 1) * dw], w1b_ref[...],
                 preferred_element_type=jnp.float32) for g in range(8)],
        axis=0)
    a_dim = avj.shape[-1]
    mj = (vj_ref[...] > 0).astype(jnp.float32)
    mw = (vw_ref[...] > 0).astype(jnp.float32)
    av3 = (base[:, None, :]
           + mj[:, :, None] * avj.reshape(bn, k, a_dim)
           + mw[:, :, None] * avw.reshape(bn, k, a_dim))
    r = jnp.maximum(av3, 0.0)
    x = jnp.sum(r * v_ref[...][None], axis=2)
    x = x - jnp.max(x, axis=1, keepdims=True)
    e = jnp.exp(x)
    a = e / jnp.sum(e, axis=1, keepdims=True)
    am = a * mj
    gj3 = gj.reshape(bn, k, f_dim)
    out_ref[...] = jnp.sum(am[:, :, None] * gj3, axis=1)


def kernel(ev, ej, ew, v_j, v_w, W_1, W_2, b, v):
    n, f_dim = ev.shape
    k = v_j.shape[1]
    dw_dim = ew.shape[1]
    a_dim = W_1.shape[1]
    e_rows = n * k
    pack = 128 // dw_dim

    w1a = W_1[:f_dim]
    w1b = W_1[f_dim:]
    bn = 400
    assert n % bn == 0
    nblk = n // bn
    rb = bn * k

    info = plsc.get_sparse_core_info()
    nw = info.num_cores * info.num_subcores
    n_chunks = 5
    nc = n // n_chunks
    ec = nc * k
    assert ec % nw == 0
    per_w = ec // nw
    c_rows = 80
    assert per_w % c_rows == 0
    t_steps = per_w // c_rows
    assert nc % bn == 0
    nblk_c = nc // bn

    jc = jnp.maximum(v_j - 1, 0)
    wc = jnp.maximum(v_w - 1, 0)
    wcp = jnp.transpose(wc.reshape(n // bn, pack, rb // pack), (0, 2, 1))

    mesh = plsc.VectorSubcoreMesh(core_axis_name="c", subcore_axis_name="s")
    gather = pl.kernel(
        functools.partial(_gather_body, (per_w, t_steps, c_rows)),
        out_type=[jax.ShapeDtypeStruct((ec, f_dim), jnp.float32),
                  jax.ShapeDtypeStruct((ec, dw_dim), jnp.float32)],
        mesh=mesh,
        scratch_types=[
            pltpu.VMEM((t_steps, c_rows), jnp.int32),
            pltpu.VMEM((t_steps, c_rows), jnp.int32),
            pltpu.VMEM((c_rows, f_dim), jnp.float32),
            pltpu.VMEM((c_rows, f_dim), jnp.float32),
            pltpu.VMEM((c_rows, dw_dim), jnp.float32),
            pltpu.VMEM((c_rows, dw_dim), jnp.float32),
            pltpu.SemaphoreType.DMA,
            pltpu.SemaphoreType.DMA,
            pltpu.SemaphoreType.DMA,
            pltpu.SemaphoreType.DMA,
        ],
        compiler_params=pltpu.CompilerParams(use_tc_tiling_on_sc=False),
    )

    attn = pl.pallas_call(
        functools.partial(_attn_body, bn, k),
        grid=(nblk_c,),
        in_specs=[
            pl.BlockSpec((rb, f_dim), lambda i: (i, 0)),
            pl.BlockSpec((rb // pack, 128), lambda i: (i, 0)),
            pl.BlockSpec((bn, k), lambda i: (i, 0)),
            pl.BlockSpec((bn, k), lambda i: (i, 0)),
            pl.BlockSpec((bn, f_dim), lambda i: (i, 0)),
            pl.BlockSpec((f_dim, a_dim), lambda i: (0, 0)),
            pl.BlockSpec((dw_dim, a_dim), lambda i: (0, 0)),
            pl.BlockSpec((f_dim, a_dim), lambda i: (0, 0)),
            pl.BlockSpec((1, a_dim), lambda i: (0, 0)),
            pl.BlockSpec((1, a_dim), lambda i: (0, 0)),
        ],
        out_specs=pl.BlockSpec((bn, f_dim), lambda i: (i, 0)),
        out_shape=jax.ShapeDtypeStruct((nc, f_dim), jnp.float32),
    )

    gathered = []
    for s in range(n_chunks):
        nd = slice(s * nc, (s + 1) * nc)
        idxj_s = jc[nd].reshape(nw, t_steps, c_rows)
        idxw_s = wcp[s * nblk_c:(s + 1) * nblk_c].reshape(nw, t_steps, c_rows)
        gathered.append(gather(ej, ew, idxj_s, idxw_s))
    outs = []
    for s in range(n_chunks):
        nd = slice(s * nc, (s + 1) * nc)
        gj, gw = gathered[s]
        gwp = gw.reshape(ec // pack, 128)
        outs.append(attn(gj, gwp, v_j[nd], v_w[nd], ev[nd],
                         w1a, w1b, W_2, b, v))
    return jnp.concatenate(outs, axis=0)
